# baseline (speedup 1.0000x reference)
import jax
import jax.numpy as jnp
from jax.experimental import pallas as pl
from jax.experimental.pallas import tpu as pltpu

_H = 32
_W = 32
_HW = _H * _W
_HALF = 384
_DIM = 2 * _HALF


def _pos_body(row_ref, col_ref, pos_ref):
    f = jax.lax.broadcasted_iota(jnp.int32, (_H, _HW), 1)
    k = jax.lax.broadcasted_iota(jnp.int32, (_H, _HW), 0)
    m_row = (f // _W == k).astype(jnp.float32)
    m_col = (f % _W == k).astype(jnp.float32)
    dn = (((0,), (0,)), ((), ()))
    pos_ref[:_HALF, :] = jax.lax.dot_general(
        row_ref[...], m_row, dn, preferred_element_type=jnp.float32)
    pos_ref[_HALF:, :] = jax.lax.dot_general(
        col_ref[...], m_col, dn, preferred_element_type=jnp.float32)


def _add_body(x_ref, pos_ref, o_ref):
    o_ref[0] = x_ref[0] + pos_ref[...]


def kernel(x, row_table, col_table):
    n, c, h, w = x.shape
    xr = x.reshape(n, c, h * w)
    pos = pl.pallas_call(
        _pos_body,
        out_shape=jax.ShapeDtypeStruct((c, h * w), jnp.float32),
    )(row_table, col_table)
    out = pl.pallas_call(
        _add_body,
        grid=(n,),
        in_specs=[
            pl.BlockSpec((1, c, h * w), lambda b: (b, 0, 0)),
            pl.BlockSpec((c, h * w), lambda b: (0, 0)),
        ],
        out_specs=pl.BlockSpec((1, c, h * w), lambda b: (b, 0, 0)),
        out_shape=jax.ShapeDtypeStruct((n, c, h * w), x.dtype),
        compiler_params=pltpu.CompilerParams(
            dimension_semantics=("parallel",)),
    )(xr, pos)
    return out.reshape(n, c, h, w)


# speedup vs baseline: 1.0105x; 1.0105x over previous
import jax
import jax.numpy as jnp
from jax.experimental import pallas as pl
from jax.experimental.pallas import tpu as pltpu

_H = 32
_W = 32
_HW = _H * _W
_HALF = 384
_DIM = 2 * _HALF


def _pos_body(row_ref, col_ref, pos_ref):
    f = jax.lax.broadcasted_iota(jnp.int32, (_H, _HW), 1)
    k = jax.lax.broadcasted_iota(jnp.int32, (_H, _HW), 0)
    m_row = (f // _W == k).astype(jnp.float32)
    m_col = (f % _W == k).astype(jnp.float32)
    dn = (((0,), (0,)), ((), ()))
    pos_ref[:_HALF, :] = jax.lax.dot_general(
        row_ref[...], m_row, dn, preferred_element_type=jnp.float32)
    pos_ref[_HALF:, :] = jax.lax.dot_general(
        col_ref[...], m_col, dn, preferred_element_type=jnp.float32)


def _add_body(x_ref, pos_ref, o_ref):
    o_ref[...] = x_ref[...] + pos_ref[...][None]


def kernel(x, row_table, col_table):
    n, c, h, w = x.shape
    xr = x.reshape(n, c, h * w)
    pos = pl.pallas_call(
        _pos_body,
        out_shape=jax.ShapeDtypeStruct((c, h * w), jnp.float32),
    )(row_table, col_table)
    nb = 4
    out = pl.pallas_call(
        _add_body,
        grid=(n // nb,),
        in_specs=[
            pl.BlockSpec((nb, c, h * w), lambda b: (b, 0, 0)),
            pl.BlockSpec((c, h * w), lambda b: (0, 0)),
        ],
        out_specs=pl.BlockSpec((nb, c, h * w), lambda b: (b, 0, 0)),
        out_shape=jax.ShapeDtypeStruct((n, c, h * w), x.dtype),
        compiler_params=pltpu.CompilerParams(
            dimension_semantics=("parallel",)),
    )(xr, pos)
    return out.reshape(n, c, h, w)


# speedup vs baseline: 3.6302x; 3.5924x over previous
import jax
import jax.numpy as jnp
from jax.experimental import pallas as pl
from jax.experimental.pallas import tpu as pltpu

_H = 32
_W = 32
_HW = _H * _W
_HALF = 384
_DIM = 2 * _HALF


def _pos_body(row_ref, col_ref, pos_ref):
    f = jax.lax.broadcasted_iota(jnp.int32, (_HW, _H), 0)
    k = jax.lax.broadcasted_iota(jnp.int32, (_HW, _H), 1)
    m_row = (f // _W == k).astype(jnp.float32)
    m_col = (f % _W == k).astype(jnp.float32)
    dn = (((1,), (0,)), ((), ()))
    pos_ref[:, :_HALF] = jax.lax.dot_general(
        m_row, row_ref[...], dn, preferred_element_type=jnp.float32)
    pos_ref[:, _HALF:] = jax.lax.dot_general(
        m_col, col_ref[...], dn, preferred_element_type=jnp.float32)


def _add_body(x_ref, pos_ref, o_ref):
    o_ref[...] = x_ref[...] + pos_ref[...][None]


def kernel(x, row_table, col_table):
    n, c, h, w = x.shape
    xt = jnp.transpose(x, (0, 2, 3, 1)).reshape(n, h * w, c)
    pos = pl.pallas_call(
        _pos_body,
        out_shape=jax.ShapeDtypeStruct((h * w, c), jnp.float32),
    )(row_table, col_table)
    nb = 2
    out = pl.pallas_call(
        _add_body,
        grid=(n // nb,),
        in_specs=[
            pl.BlockSpec((nb, h * w, c), lambda b: (b, 0, 0)),
            pl.BlockSpec((h * w, c), lambda b: (0, 0)),
        ],
        out_specs=pl.BlockSpec((nb, h * w, c), lambda b: (b, 0, 0)),
        out_shape=jax.ShapeDtypeStruct((n, h * w, c), x.dtype),
        compiler_params=pltpu.CompilerParams(
            dimension_semantics=("parallel",)),
    )(xt, pos)
    return jnp.transpose(out.reshape(n, h, w, c), (0, 3, 1, 2))


# speedup vs baseline: 3.8160x; 1.0512x over previous
import jax
import jax.numpy as jnp
from jax.experimental import pallas as pl
from jax.experimental.pallas import tpu as pltpu

_H = 32
_W = 32
_HW = _H * _W
_HALF = 384
_DIM = 2 * _HALF


def _body(x_ref, row_ref, col_ref, o_ref, pos_ref):
    @pl.when(pl.program_id(0) == 0)
    def _init_pos():
        f = jax.lax.broadcasted_iota(jnp.int32, (_HW, _H), 0)
        k = jax.lax.broadcasted_iota(jnp.int32, (_HW, _H), 1)
        m_row = (f // _W == k).astype(jnp.float32)
        m_col = (f % _W == k).astype(jnp.float32)
        dn = (((1,), (0,)), ((), ()))
        pos_ref[:, :_HALF] = jax.lax.dot_general(
            m_row, row_ref[...], dn, preferred_element_type=jnp.float32)
        pos_ref[:, _HALF:] = jax.lax.dot_general(
            m_col, col_ref[...], dn, preferred_element_type=jnp.float32)

    o_ref[...] = x_ref[...] + pos_ref[...][None]


def kernel(x, row_table, col_table):
    n, c, h, w = x.shape
    xt = jnp.transpose(x, (0, 2, 3, 1)).reshape(n, h * w, c)
    nb = 2
    out = pl.pallas_call(
        _body,
        grid=(n // nb,),
        in_specs=[
            pl.BlockSpec((nb, h * w, c), lambda b: (b, 0, 0)),
            pl.BlockSpec((h, _HALF), lambda b: (0, 0)),
            pl.BlockSpec((w, _HALF), lambda b: (0, 0)),
        ],
        out_specs=pl.BlockSpec((nb, h * w, c), lambda b: (b, 0, 0)),
        out_shape=jax.ShapeDtypeStruct((n, h * w, c), x.dtype),
        scratch_shapes=[pltpu.VMEM((h * w, c), jnp.float32)],
    )(xt, row_table, col_table)
    return jnp.transpose(out.reshape(n, h, w, c), (0, 3, 1, 2))


# speedup vs baseline: 3.9647x; 1.0390x over previous
import jax
import jax.numpy as jnp
from jax.experimental import pallas as pl
from jax.experimental.pallas import tpu as pltpu

_H = 32
_W = 32
_HW = _H * _W
_HALF = 384
_DIM = 2 * _HALF


def _body(x_ref, row_ref, col_ref, o_ref, pos_ref):
    @pl.when(pl.program_id(0) == 0)
    def _init_pos():
        f = jax.lax.broadcasted_iota(jnp.int32, (_HW, _H), 0)
        k = jax.lax.broadcasted_iota(jnp.int32, (_HW, _H), 1)
        m_row = (f // _W == k).astype(jnp.float32)
        m_col = (f % _W == k).astype(jnp.float32)
        dn = (((1,), (0,)), ((), ()))
        pos_ref[:, :_HALF] = jax.lax.dot_general(
            m_row, row_ref[...], dn, preferred_element_type=jnp.float32)
        pos_ref[:, _HALF:] = jax.lax.dot_general(
            m_col, col_ref[...], dn, preferred_element_type=jnp.float32)

    o_ref[...] = x_ref[...] + pos_ref[...][None]


def kernel(x, row_table, col_table):
    n, c, h, w = x.shape
    xt = jnp.transpose(x, (0, 2, 3, 1)).reshape(n, h * w, c)
    nb = 4
    out = pl.pallas_call(
        _body,
        grid=(n // nb,),
        in_specs=[
            pl.BlockSpec((nb, h * w, c), lambda b: (b, 0, 0)),
            pl.BlockSpec((h, _HALF), lambda b: (0, 0)),
            pl.BlockSpec((w, _HALF), lambda b: (0, 0)),
        ],
        out_specs=pl.BlockSpec((nb, h * w, c), lambda b: (b, 0, 0)),
        out_shape=jax.ShapeDtypeStruct((n, h * w, c), x.dtype),
        scratch_shapes=[pltpu.VMEM((h * w, c), jnp.float32)],
    )(xt, row_table, col_table)
    return jnp.transpose(out.reshape(n, h, w, c), (0, 3, 1, 2))
